# baseline (speedup 1.0000x reference)
import functools

import jax
import jax.numpy as jnp
from jax import lax
from jax.experimental import pallas as pl
from jax.experimental.pallas import tpu as pltpu
from jax.experimental.pallas import tpu_sc as plsc

B, Q, C, T, L = 16, 4096, 128, 64, 6
QH = Q // 2
CH = 128
NCHK = QH // CH
NG = T // 16


def _lane_shuffle(x, idx):
    return lax.gather(
        x, idx[:, None],
        lax.GatherDimensionNumbers(
            offset_dims=(), collapsed_slice_dims=(0,), start_index_map=(0,)),
        (1,), mode=lax.GatherScatterMode.PROMISE_IN_BOUNDS)


def _sc_kernel_body(logits_hbm, pn_hbm, tgt_hbm, bq_out, keep_out, ms_out,
                    buf0, buf1, pnbuf, lblbuf, v64, i64, k64, stage, pstage,
                    shv, sem0, sem1):
    cc = lax.axis_index("c")
    ss = lax.axis_index("s")
    b = cc * 8 + ss // 2
    half = ss % 2
    qlo = half * QH

    pltpu.sync_copy(tgt_hbm.at[b], lblbuf)
    pltpu.sync_copy(pn_hbm.at[:, b, pl.ds(qlo * 2, QH * 2)], pnbuf)
    lbls = [lblbuf[pl.ds(k * 16, 16)] for k in range(NG)]

    flat0 = (b * Q + qlo) * C

    lane_iota = lax.iota(jnp.int32, 16)
    perms = [lane_iota ^ m for m in (8, 4, 2, 1)]

    def make_row_body(buf, ck):
        def row_body(r, carry):
            bvs, bis = carry[:NG], carry[NG:]
            base = pl.multiple_of(r * C, C)
            ex = None
            for i in range(C // 16):
                e_i = jnp.exp(buf[pl.ds(base + i * 16, 16)])
                ex = e_i if ex is None else ex + e_i
            for pm in perms:
                ex = ex + _lane_shuffle(ex, pm)
            qsplat = jnp.full((16,), qlo + ck * CH, jnp.int32) + r
            nbvs, nbis = [], []
            for k in range(NG):
                g = plsc.load_gather(buf, [lbls[k] + base])
                p = jnp.exp(g) / ex
                upd = p > bvs[k]
                nbvs.append(jnp.where(upd, p, bvs[k]))
                nbis.append(jnp.where(upd, qsplat, bis[k]))
            return tuple(nbvs) + tuple(nbis)
        return row_body

    bvs = [jnp.full((16,), -jnp.inf, jnp.float32) for _ in range(NG)]
    bis = [jnp.zeros((16,), jnp.int32) for _ in range(NG)]
    bufs = (buf0, buf1)
    sems = (sem0, sem1)
    cps = [None] * NCHK
    cps[0] = pltpu.async_copy(
        logits_hbm.at[pl.ds(flat0, CH * C)], buf0, sem0)
    for ck in range(NCHK):
        if ck + 1 < NCHK:
            cps[ck + 1] = pltpu.async_copy(
                logits_hbm.at[pl.ds(flat0 + (ck + 1) * CH * C, CH * C)],
                bufs[(ck + 1) % 2], sems[(ck + 1) % 2])
        cps[ck].wait()
        carry = lax.fori_loop(
            0, CH, make_row_body(bufs[ck % 2], ck), tuple(bvs) + tuple(bis))
        bvs = list(carry[:NG])
        bis = list(carry[NG:])

    avgs = []
    for k in range(NG):
        qloc2 = (bis[k] - qlo) * 2 + 1
        t01 = (plsc.load_gather(pnbuf, [jnp.full((16,), 0, jnp.int32), qloc2])
               + plsc.load_gather(pnbuf, [jnp.full((16,), 1, jnp.int32), qloc2]))
        t23 = (plsc.load_gather(pnbuf, [jnp.full((16,), 2, jnp.int32), qloc2])
               + plsc.load_gather(pnbuf, [jnp.full((16,), 3, jnp.int32), qloc2]))
        t45 = (plsc.load_gather(pnbuf, [jnp.full((16,), 4, jnp.int32), qloc2])
               + plsc.load_gather(pnbuf, [jnp.full((16,), 5, jnp.int32), qloc2]))
        avgs.append(((t01 + t23) + t45) / 6.0)

    for k in range(NG):
        stage[pl.ds(k * 16, 16)] = bvs[k]
        stage[pl.ds(T + k * 16, 16)] = avgs[k]
        stage[pl.ds(2 * T + k * 16, 16)] = bis[k].astype(jnp.float32)
    pltpu.sync_copy(stage, shv.at[ss])
    plsc.subcore_barrier()

    @pl.when(half == 0)
    def _merge():
        pltpu.sync_copy(shv.at[ss + 1], pstage)
        masks = []
        midxs = []
        mavgs = []
        for k in range(NG):
            pv = pstage[pl.ds(k * 16, 16)]
            pa = pstage[pl.ds(T + k * 16, 16)]
            pi = pstage[pl.ds(2 * T + k * 16, 16)].astype(jnp.int32)
            upd = pv > bvs[k]
            midxs.append(jnp.where(upd, pi, bis[k]))
            mavgs.append(jnp.where(upd, pa, avgs[k]))
            masks.append(mavgs[k] > 0.3)
        total = (plsc.all_reduce_population_count(masks[0])
                 + plsc.all_reduce_population_count(masks[1])
                 + plsc.all_reduce_population_count(masks[2])
                 + plsc.all_reduce_population_count(masks[3]))
        anyh = total > 0
        ones = jnp.full((16,), 1, jnp.int32)
        for k in range(NG):
            v64[pl.ds(k * 16, 16)] = mavgs[k]
            i64[pl.ds(k * 16, 16)] = midxs[k]
            k64[pl.ds(k * 16, 16)] = jnp.where(
                anyh, masks[k].astype(jnp.int32), ones)
        pltpu.sync_copy(i64, bq_out.at[b])
        pltpu.sync_copy(k64, keep_out.at[b])
        pltpu.sync_copy(v64, ms_out.at[b])


_sc_call = functools.partial(
    pl.kernel,
    mesh=plsc.VectorSubcoreMesh(core_axis_name="c", subcore_axis_name="s"),
    compiler_params=pltpu.CompilerParams(needs_layout_passes=False),
    out_type=[
        jax.ShapeDtypeStruct((B, T), jnp.int32),
        jax.ShapeDtypeStruct((B, T), jnp.int32),
        jax.ShapeDtypeStruct((B, T), jnp.float32),
    ],
    scratch_types=[
        pltpu.VMEM((CH * C,), jnp.float32),
        pltpu.VMEM((CH * C,), jnp.float32),
        pltpu.VMEM((L, QH * 2), jnp.float32),
        pltpu.VMEM((T,), jnp.int32),
        pltpu.VMEM((T,), jnp.float32),
        pltpu.VMEM((T,), jnp.int32),
        pltpu.VMEM((T,), jnp.int32),
        pltpu.VMEM((4 * T,), jnp.float32),
        pltpu.VMEM((4 * T,), jnp.float32),
        pltpu.VMEM_SHARED((16, 4 * T), jnp.float32),
        pltpu.SemaphoreType.DMA,
        pltpu.SemaphoreType.DMA,
    ],
)(_sc_kernel_body)


def kernel(pred_logits, pos_neg_probs, tgt_labels):
    logits_flat = pred_logits.reshape(B * Q * C)
    pn3 = pos_neg_probs.reshape(L, B, Q * 2)
    bq, keep, ms = _sc_call(logits_flat, pn3, tgt_labels.astype(jnp.int32))
    base_target_idx = jnp.broadcast_to(
        jnp.arange(T, dtype=tgt_labels.dtype)[None, :], (B, T))
    return (bq, base_target_idx, keep.astype(jnp.bool_), ms)
